# Initial kernel scaffold; baseline (speedup 1.0000x reference)
import jax
import jax.numpy as jnp
from jax.experimental import pallas as pl

_HI = jax.lax.Precision.HIGHEST


def _mm(a, b):
    return jnp.matmul(a, b, precision=_HI)


def kernel(feature, knn_idx, Qw0, Kw0, Vw0, Qb0, Kb0, Vb0, g0, b0,
           Qw1, Kw1, Vw1, Qb1, Kb1, Vb1, g1, b1,
           Qw2, Kw2, Vw2, Qb2, Kb2, Vb2, g2, b2):
    B, C, N = feature.shape
    K = knn_idx.shape[-1]
    X = jnp.transpose(feature, (0, 2, 1))

    neigh = jax.vmap(lambda x, idx: jnp.take(x, idx, axis=0))(X, knn_idx)
    sim = jnp.einsum('bnc,bnkc->bnk', X, neigh, precision=_HI)
    att = jax.nn.softmax(sim, axis=-1)
    i1 = jnp.argmax(att, axis=-1)
    lane = jnp.arange(K)
    att2 = jnp.where(lane[None, None, :] == i1[..., None], -jnp.inf, att)
    s = jnp.argmax(att2, axis=-1).astype(jnp.int32)

    f = s[0, :16]
    F = (f[:, None] == jnp.arange(16)[None, :]).astype(jnp.float32)
    O = jnp.eye(16, dtype=jnp.float32)
    rows = [O]
    for _ in range(15):
        O = _mm(O, F)
        rows.append(O)
    onehotW = jnp.concatenate(rows, axis=0)

    sb = s.reshape(B * N)
    h = X.reshape(B * N, C)
    lane256 = jnp.arange(256)
    smod = lane256 % 16
    mask_sw = (smod[:, None] == smod[None, :])
    NEG = jnp.float32(-jnp.inf)

    outs = []
    weights = [(Qw0, Kw0, Vw0, Qb0, Kb0, Vb0, g0, b0),
               (Qw1, Kw1, Vw1, Qb1, Kb1, Vb1, g1, b1),
               (Qw2, Kw2, Vw2, Qb2, Kb2, Vb2, g2, b2)]
    for (Qw, Kw, Vw, Qb, Kb, Vb, g, bb) in weights:
        h16 = jnp.stack([h[0:16], h[N:N + 16]], axis=0)
        Wtok = _mm(onehotW[None], h16)
        Qt = _mm(Wtok, Qw.T) + Qb
        Kt = _mm(Wtok, Kw.T) + Kb
        Vt = _mm(Wtok, Vw.T) + Vb
        S = _mm(Qt, jnp.swapaxes(Kt, -1, -2))
        Sm = jnp.where(mask_sw[None], S, NEG)
        rm = Sm.max(-1)
        E = jnp.where(mask_sw[None], jnp.exp(S - rm[..., None]), 0.0).sum(-1)
        Pdg = jnp.exp(jnp.diagonal(S, axis1=-2, axis2=-1) - rm)

        Qc = _mm(h, Qw.T) + Qb
        Kc = _mm(h, Kw.T) + Kb
        Vc = _mm(h, Vw.T) + Vb

        xv_parts = []
        for b in range(B):
            sl = slice(b * N, (b + 1) * N)
            Kc_b, Qc_b, Vc_b = Kc[sl], Qc[sl], Vc[sl]
            a = _mm(Kc_b, Qt[b].T)
            mx = jnp.maximum(rm[b][None, :], a)
            t = jnp.exp(rm[b][None, :] - mx)
            u = jnp.exp(a - mx)
            attw = Pdg[b][None, :] * t / (E[b][None, :] * t + u)
            oneh = (smod[None, :] == sb[sl][:, None])
            wvec = jnp.where(oneh, attw, 0.0)
            xv_w = _mm(wvec, Vt[b])
            M2 = _mm(Qc_b, Kt[b].T)
            Bm = jnp.where(oneh, M2, NEG)
            bmax = Bm.max(-1)
            D = (Qc_b * Kc_b).sum(-1)
            m = jnp.maximum(bmax, D)
            sume = jnp.where(oneh, jnp.exp(M2 - m[:, None]), 0.0).sum(-1)
            eD = jnp.exp(D - m)
            attc = eD / (sume + eD)
            xv_parts.append(xv_w + attc[:, None] * Vc_b)
        xv = jnp.concatenate(xv_parts, axis=0)

        mean = xv.mean(axis=0)
        var = xv.var(axis=0)
        xn = (xv - mean) / jnp.sqrt(var + 1e-5) * g + bb
        h = h + jax.nn.relu(xn)
        outs.append(h)

    out = jnp.concatenate(outs, axis=-1)
    return jnp.transpose(out.reshape(B, N, 3 * C), (0, 2, 1))


# speedup vs baseline: 4.5913x; 4.5913x over previous
import jax
import jax.numpy as jnp
from jax.experimental import pallas as pl

B, C, N, K = 2, 256, 4096, 16
M = B * N
NB = 8
RB = M // NB
_HI = jax.lax.Precision.HIGHEST
_BF = jnp.bfloat16
_NEG = float('-inf')


def _dotg(a, b, dims, precision=None):
    return jax.lax.dot_general(a, b, (dims, ((), ())),
                               preferred_element_type=jnp.float32,
                               precision=precision)


def _dot_split(a, b):
    p1 = _dotg(a[:, :128], b[:, :128], ((1,), (1,)), precision=_HI)
    p2 = _dotg(a[:, 128:], b[:, 128:], ((1,), (1,)), precision=_HI)
    return _rbf(p1) + _rbf(p2)


def _rbf(x):
    b = jax.lax.bitcast_convert_type(x, jnp.uint32)
    r = (b + jnp.uint32(0x7FFF) + ((b >> 16) & jnp.uint32(1))) & jnp.uint32(0xFFFF0000)
    return jax.lax.bitcast_convert_type(r, jnp.float32)


def _a2_body(sim_ref, o_ref):
    s = sim_ref[...]
    iota = jax.lax.broadcasted_iota(jnp.int32, s.shape, 1).astype(jnp.float32)
    m1 = jnp.max(s, axis=1, keepdims=True)
    i1 = jnp.min(jnp.where(s == m1, iota, 16.0), axis=1, keepdims=True)
    s2 = jnp.where(iota == i1, _NEG, s)
    m2 = jnp.max(s2, axis=1, keepdims=True)
    o_ref[...] = jnp.min(jnp.where(s2 == m2, iota, 16.0), axis=1, keepdims=True)


def _prep_body(a2_ref, h16_ref, qw, kw, vw, qb, kb, vb,
               qt_o, kt_o, vt_o, rm_o, e_o, p_o):
    f_col = a2_ref[...]
    lane16 = jax.lax.broadcasted_iota(jnp.int32, (16, 16), 1).astype(jnp.float32)
    F = (f_col == lane16).astype(jnp.float32)
    h16 = h16_ref[0]

    Ok = (jax.lax.broadcasted_iota(jnp.int32, (16, 16), 0) ==
          jax.lax.broadcasted_iota(jnp.int32, (16, 16), 1)).astype(jnp.float32)
    wtok_rows = []
    for k in range(16):
        acc = jnp.zeros((16, C), jnp.float32)
        for j in range(16):
            acc = acc + Ok[:, j:j + 1] * h16[j:j + 1, :]
        wtok_rows.append(acc)
        if k < 15:
            Ok = jnp.dot(Ok, F, preferred_element_type=jnp.float32)
    wtok = jnp.concatenate(wtok_rows, axis=0)

    wb = _rbf(wtok)
    Qt = _dotg(wb, qw[...], ((1,), (1,)), precision=_HI) + qb[...]
    Kt = _dotg(wb, kw[...], ((1,), (1,)), precision=_HI) + kb[...]
    Vt = _dotg(wb, vw[...], ((1,), (1,)), precision=_HI) + vb[...]

    St = _dotg(_rbf(Kt), _rbf(Qt), ((1,), (1,)), precision=_HI)
    r0 = jax.lax.broadcasted_iota(jnp.int32, (256, 256), 0)
    c0 = jax.lax.broadcasted_iota(jnp.int32, (256, 256), 1)
    mask = jnp.bitwise_and(r0, 15) == jnp.bitwise_and(c0, 15)
    Stm = jnp.where(mask, St, _NEG)
    rm = jnp.max(Stm, axis=0, keepdims=True)
    E = jnp.sum(jnp.where(mask, jnp.exp(St - rm), 0.0), axis=0, keepdims=True)
    diag = jnp.sum(jnp.where(r0 == c0, St, 0.0), axis=0, keepdims=True)
    P = jnp.exp(diag - rm)

    qt_o[0] = Qt
    kt_o[0] = Kt
    vt_o[0] = Vt
    rm_o[0] = rm
    e_o[0] = E
    p_o[0] = P


def _main_body(h_ref, s_ref, qw, kw, vw, qb, kb, vb,
               qt, kt, vt, rm, e, p, xv_o, sum_o, ssq_o):
    h = h_ref[...]
    hb = _rbf(h)
    Qc = _dotg(hb, qw[...], ((1,), (1,)), precision=_HI) + qb[...]
    Kc = _dotg(hb, kw[...], ((1,), (1,)), precision=_HI) + kb[...]
    Vc = _dotg(hb, vw[...], ((1,), (1,)), precision=_HI) + vb[...]

    A = _dotg(_rbf(Kc), _rbf(qt[0]), ((1,), (1,)), precision=_HI)
    M2 = _dotg(_rbf(Qc), _rbf(kt[0]), ((1,), (1,)), precision=_HI)
    D = jnp.sum(_rbf(Qc) * _rbf(Kc), axis=1, keepdims=True)

    rmv, ev, pv = rm[0], e[0], p[0]
    mx = jnp.maximum(rmv, A)
    t = jnp.exp(rmv - mx)
    u = jnp.exp(A - mx)
    attw = pv * t / (ev * t + u)

    lane = jax.lax.broadcasted_iota(jnp.int32, (RB, 256), 1)
    smod = jnp.bitwise_and(lane, 15).astype(jnp.float32)
    oneh = smod == s_ref[...]
    wvec = jnp.where(oneh, attw, 0.0)
    xvw = _dotg(wvec, vt[0], ((1,), (0,)), precision=_HI)

    Bm = jnp.where(oneh, M2, _NEG)
    bmax = jnp.max(Bm, axis=1, keepdims=True)
    mm = jnp.maximum(bmax, D)
    sume = jnp.sum(jnp.where(oneh, jnp.exp(M2 - mm), 0.0), axis=1, keepdims=True)
    eD = jnp.exp(D - mm)
    attc = eD / (sume + eD)

    xv = xvw + attc * Vc
    xv_o[...] = xv
    ps = jnp.sum(xv, axis=0, keepdims=True)
    pq = jnp.sum(xv * xv, axis=0, keepdims=True)

    @pl.when(pl.program_id(0) == 0)
    def _():
        sum_o[...] = ps
        ssq_o[...] = pq

    @pl.when(pl.program_id(0) != 0)
    def _():
        sum_o[...] += ps
        ssq_o[...] += pq


def _bn_body(xv_ref, h_ref, sum_ref, ssq_ref, g_ref, b_ref, o_ref):
    mean = sum_ref[...] / M
    var = ssq_ref[...] / M - mean * mean
    inv = 1.0 / jnp.sqrt(var + 1e-5)
    xn = (xv_ref[...] - mean) * inv * g_ref[...] + b_ref[...]
    o_ref[...] = h_ref[...] + jnp.maximum(xn, 0.0)


def _row_spec():
    return pl.BlockSpec((RB, C), lambda i: (i, 0))


def kernel(feature, knn_idx, Qw0, Kw0, Vw0, Qb0, Kb0, Vb0, g0, b0,
           Qw1, Kw1, Vw1, Qb1, Kb1, Vb1, g1, b1,
           Qw2, Kw2, Vw2, Qb2, Kb2, Vb2, g2, b2):
    X = jnp.transpose(feature, (0, 2, 1))

    neigh = jax.vmap(lambda f, idx: jnp.take(f, idx, axis=1))(feature, knn_idx)
    center = jnp.transpose(feature[..., None], (0, 2, 3, 1))
    sim = jnp.matmul(center, jnp.transpose(neigh, (0, 2, 1, 3)))[:, :, 0, :]
    att = jax.nn.softmax(sim, axis=-1)
    sim_flat = att.reshape(M, K)

    s_col = pl.pallas_call(
        _a2_body,
        grid=(NB,),
        in_specs=[pl.BlockSpec((RB, K), lambda i: (i, 0))],
        out_specs=pl.BlockSpec((RB, 1), lambda i: (i, 0)),
        out_shape=jax.ShapeDtypeStruct((M, 1), jnp.float32),
    )(sim_flat)

    a2_16 = s_col[:16]

    h = X.reshape(M, C)
    full = lambda shape: pl.BlockSpec(shape, lambda *_: tuple(0 for _ in shape))
    w_spec = full((C, C))
    b_spec = full((1, C))

    outs = []
    params = [(Qw0, Kw0, Vw0, Qb0, Kb0, Vb0, g0, b0),
              (Qw1, Kw1, Vw1, Qb1, Kb1, Vb1, g1, b1),
              (Qw2, Kw2, Vw2, Qb2, Kb2, Vb2, g2, b2)]
    for (Qw, Kw, Vw, Qb, Kb, Vb, g, bb) in params:
        Qb2d, Kb2d, Vb2d = Qb.reshape(1, C), Kb.reshape(1, C), Vb.reshape(1, C)

        Qt, Kt, Vt, rm, E, P = pl.pallas_call(
            _prep_body,
            grid=(B,),
            in_specs=[full((16, 1)),
                      pl.BlockSpec((1, 16, C), lambda b: (b, 0, 0)),
                      w_spec, w_spec, w_spec, b_spec, b_spec, b_spec],
            out_specs=[pl.BlockSpec((1, C, C), lambda b: (b, 0, 0))] * 3 +
                      [pl.BlockSpec((1, 1, C), lambda b: (b, 0, 0))] * 3,
            out_shape=[jax.ShapeDtypeStruct((B, C, C), jnp.float32)] * 3 +
                      [jax.ShapeDtypeStruct((B, 1, C), jnp.float32)] * 3,
        )(a2_16, h.reshape(B, N, C)[:, :16], Qw, Kw, Vw, Qb2d, Kb2d, Vb2d)

        tbl = lambda: pl.BlockSpec((1, C, C), lambda i: (i // (NB // B), 0, 0))
        vec = lambda: pl.BlockSpec((1, 1, C), lambda i: (i // (NB // B), 0, 0))
        xv, ssum, ssq = pl.pallas_call(
            _main_body,
            grid=(NB,),
            in_specs=[_row_spec(), pl.BlockSpec((RB, 1), lambda i: (i, 0)),
                      w_spec, w_spec, w_spec, b_spec, b_spec, b_spec,
                      tbl(), tbl(), tbl(), vec(), vec(), vec()],
            out_specs=[_row_spec(),
                       pl.BlockSpec((1, C), lambda i: (0, 0)),
                       pl.BlockSpec((1, C), lambda i: (0, 0))],
            out_shape=[jax.ShapeDtypeStruct((M, C), jnp.float32),
                       jax.ShapeDtypeStruct((1, C), jnp.float32),
                       jax.ShapeDtypeStruct((1, C), jnp.float32)],
        )(h, s_col, Qw, Kw, Vw, Qb2d, Kb2d, Vb2d, Qt, Kt, Vt, rm, E, P)

        h = pl.pallas_call(
            _bn_body,
            grid=(NB,),
            in_specs=[_row_spec(), _row_spec(), full((1, C)), full((1, C)),
                      full((1, C)), full((1, C))],
            out_specs=_row_spec(),
            out_shape=jax.ShapeDtypeStruct((M, C), jnp.float32),
        )(xv, h, ssum, ssq, g.reshape(1, C), bb.reshape(1, C))
        outs.append(h)

    out = jnp.concatenate(outs, axis=-1)
    return jnp.transpose(out.reshape(B, N, 3 * C), (0, 2, 1))
